# baseline (speedup 1.0000x reference)
import jax
import jax.numpy as jnp
from jax import lax
from jax.experimental import pallas as pl
from jax.experimental.pallas import tpu as pltpu
from jax.experimental.pallas import tpu_sc as plsc

NC = 2
NS = 16
NW = NC * NS

B, L, W = 4096, 200, 32
TOTAL = B * L
SUB = 128
NSUB = 10
CHUNK = SUB * NSUB
NCHUNKS = TOTAL // CHUNK
CPW = NCHUNKS // NW
TROWS = 2056
PAD_ROW = 2048

TC_ROWS = TOTAL // 128
TC_BLOCK = 800
TC_GRID = TC_ROWS // TC_BLOCK


def _tc_select_body(x_ref, m_ref, o_ref):
    o_ref[...] = jnp.where(m_ref[...], x_ref[...], PAD_ROW)


def _masked_indices(idx2, mask2):
    spec = pl.BlockSpec((TC_BLOCK, 128), lambda i: (i, 0))
    return pl.pallas_call(
        _tc_select_body,
        grid=(TC_GRID,),
        in_specs=[spec, spec],
        out_specs=spec,
        out_shape=jax.ShapeDtypeStruct((TC_ROWS, 128), jnp.int32),
    )(idx2, mask2)


def _sc_body(idxm_hbm, table_hbm, out_hbm,
             table_v, idxm0, idxm1, rows0, rows1,
             insem0, insem1, gsem0, gsem1, outsem0, outsem1):
    wid = lax.axis_index("s") * NC + lax.axis_index("c")
    base = wid * CPW
    idxm_bufs = (idxm0, idxm1)
    row_bufs = (rows0, rows1)
    insems = (insem0, insem1)
    gsems = (gsem0, gsem1)
    outsems = (outsem0, outsem1)

    def start_in(cid, slot):
        pltpu.async_copy(idxm_hbm.at[cid], idxm_bufs[slot], insems[slot])

    start_in(base + 0, 0)
    start_in(base + 1, 1)

    @pl.when(lax.axis_index("s") == 0)
    def _():
        pltpu.sync_copy(table_hbm, table_v)

    plsc.subcore_barrier()

    def do_chunk(c, slot):
        idxm_v = idxm_bufs[slot]
        rows_v = row_bufs[slot]
        pltpu.make_async_copy(idxm_hbm.at[0], idxm_v, insems[slot]).wait()

        @pl.when(c >= base + 2)
        def _():
            pltpu.make_async_copy(rows_v, out_hbm.at[c], outsems[slot]).wait()

        for j in range(NSUB):
            pltpu.async_copy(table_v.at[idxm_v.at[j]], rows_v.at[j],
                             gsems[slot])
        pltpu.make_async_copy(out_hbm.at[c], rows_v, gsems[slot]).wait()

        @pl.when(c + 2 < base + CPW)
        def _():
            start_in(c + 2, slot)

        pltpu.async_copy(rows_v, out_hbm.at[c], outsems[slot])

    def body(t, carry):
        do_chunk(base + 2 * t, 0)
        do_chunk(base + 2 * t + 1, 1)
        return carry

    lax.fori_loop(0, CPW // 2, body, 0)
    pltpu.make_async_copy(rows0, out_hbm.at[base], outsems[0]).wait()
    pltpu.make_async_copy(rows1, out_hbm.at[base], outsems[1]).wait()


def _sc_gather(idxm3, table_pad):
    mesh = plsc.VectorSubcoreMesh(core_axis_name="c", subcore_axis_name="s")
    kfn = pl.kernel(
        _sc_body,
        out_type=jax.ShapeDtypeStruct((NCHUNKS, NSUB, SUB, W), jnp.float32),
        mesh=mesh,
        scratch_types=[
            pltpu.VMEM_SHARED((TROWS, W), jnp.float32),
            pltpu.VMEM((NSUB, SUB), jnp.int32),
            pltpu.VMEM((NSUB, SUB), jnp.int32),
            pltpu.VMEM((NSUB, SUB, W), jnp.float32),
            pltpu.VMEM((NSUB, SUB, W), jnp.float32),
            pltpu.SemaphoreType.DMA,
            pltpu.SemaphoreType.DMA,
            pltpu.SemaphoreType.DMA,
            pltpu.SemaphoreType.DMA,
            pltpu.SemaphoreType.DMA,
            pltpu.SemaphoreType.DMA,
        ],
        compiler_params=pltpu.CompilerParams(use_tc_tiling_on_sc=False),
    )
    return kfn(idxm3, table_pad)


@jax.jit
def _run(idx2, mask2, table_pad):
    idxm = _masked_indices(idx2, mask2)
    return _sc_gather(idxm.reshape(NCHUNKS, NSUB, SUB), table_pad)


def kernel(obs_pos, obs_mask, embedding_table):
    idx2 = obs_pos.astype(jnp.int32).reshape(TC_ROWS, 128)
    mask2 = obs_mask.reshape(TC_ROWS, 128)
    table_pad = jnp.concatenate(
        [embedding_table, jnp.zeros((TROWS - 2048, W), jnp.float32)], axis=0)
    out = _run(idx2, mask2, table_pad)
    return out.reshape(B, L, W)
